# Initial kernel scaffold; baseline (speedup 1.0000x reference)
import jax
import jax.numpy as jnp
from jax.experimental import pallas as pl


def kernel(x, edge_index, edge_weight, Wg1, Wg2, Wg3, W1, b1, W3, b3, Wd, bd):
    raise NotImplementedError("write your pallas kernel here")


# speedup vs baseline: 3.6493x; 3.6493x over previous
import functools

import jax
import jax.numpy as jnp
from jax import lax
from jax.experimental import pallas as pl
from jax.experimental.pallas import tpu as pltpu
from jax.experimental.pallas import tpu_sc as plsc

N = 10000
E = 320000
D = 128
OUT = 7

NC = 2
NS = 16
NW = NC * NS
K = 128
NCH = 80
E_PAD = NW * NCH * K
RB = 80
NBLK = N // RB

_MESH = plsc.VectorSubcoreMesh(core_axis_name="c", subcore_axis_name="s")


def _sc_agg_body(s_hbm, row_hbm, col_hbm, w_hbm, zeros_hbm, p_hbm,
                 acc_sh, row_v, col_v, w_v, rows_v, sem):
    cid = lax.axis_index("c")
    sid = lax.axis_index("s")
    wid = cid * NS + sid

    pltpu.sync_copy(zeros_hbm, rows_v.at[pl.ds(0, RB)])
    for i in range((NBLK + NS - 1) // NS):
        blk = sid + i * NS

        @pl.when(blk < NBLK)
        def _zero():
            pltpu.sync_copy(rows_v.at[pl.ds(0, RB)], acc_sh.at[pl.ds(blk * RB, RB)])

    pltpu.sync_copy(row_hbm.at[wid], row_v)
    pltpu.sync_copy(col_hbm.at[wid], col_v)
    pltpu.sync_copy(w_hbm.at[wid], w_v)
    plsc.subcore_barrier()

    def chunk(c, carry):
        pltpu.async_copy(s_hbm.at[col_v.at[c]], rows_v, sem).wait()

        def scale(g, carry2):
            wv = w_v[c, pl.ds(g * 16, 16)]
            base = g * 16
            for j in range(16):
                wk = wv[j]
                for d in range(D // 16):
                    sl = pl.ds(d * 16, 16)
                    rows_v[base + j, sl] = rows_v[base + j, sl] * wk
            return carry2
        lax.fori_loop(0, K // 16, scale, 0)

        pltpu.sync_copy(rows_v, acc_sh.at[row_v.at[c]], add=True)
        return carry
    lax.fori_loop(0, NCH, chunk, 0)
    plsc.subcore_barrier()

    for i in range((NBLK + NS - 1) // NS):
        blk = sid + i * NS

        @pl.when(blk < NBLK)
        def _writeback():
            r0 = blk * RB
            pltpu.sync_copy(acc_sh.at[pl.ds(r0, RB)], rows_v.at[pl.ds(0, RB)])
            pltpu.sync_copy(rows_v.at[pl.ds(0, RB)], p_hbm.at[cid, pl.ds(r0, RB)])


_sc_agg = functools.partial(
    pl.kernel,
    out_type=jax.ShapeDtypeStruct((NC, N, D), jnp.float32),
    mesh=_MESH,
    scratch_types=[
        pltpu.VMEM_SHARED((N, D), jnp.float32),
        pltpu.VMEM((NCH, K), jnp.int32),
        pltpu.VMEM((NCH, K), jnp.int32),
        pltpu.VMEM((NCH, K), jnp.float32),
        pltpu.VMEM((K, D), jnp.float32),
        pltpu.SemaphoreType.DMA,
    ],
)(_sc_agg_body)


def _tc_mid_body(p_ref, x_ref, wg1_ref, wg3_ref, w1_ref, b1_ref,
                 h1_ref, hg2_ref, hl_ref):
    axc = p_ref[0] + p_ref[1]
    h1_ref[...] = jnp.maximum(
        jnp.dot(axc, wg1_ref[...], preferred_element_type=jnp.float32), 0.0)
    hg2_ref[...] = jnp.maximum(
        jnp.dot(axc, wg3_ref[...], preferred_element_type=jnp.float32), 0.0)
    hl_ref[...] = jnp.maximum(
        jnp.dot(x_ref[...], w1_ref[...], preferred_element_type=jnp.float32)
        + b1_ref[...], 0.0)


def _tc_final_body(p_ref, hg2_ref, hl_ref, wg2_ref, w3_ref, b3_ref,
                   wdp_ref, bdp_ref, out_ref):
    ahc = p_ref[0] + p_ref[1]
    hg1 = jnp.maximum(
        jnp.dot(ahc, wg2_ref[...], preferred_element_type=jnp.float32), 0.0)
    z = (jnp.dot(hg1, w3_ref[0:D, :], preferred_element_type=jnp.float32)
         + jnp.dot(hg2_ref[...], w3_ref[D:2 * D, :],
                   preferred_element_type=jnp.float32)
         + jnp.dot(hl_ref[...], w3_ref[2 * D:3 * D, :],
                   preferred_element_type=jnp.float32)
         + b3_ref[...])
    c = jnp.dot(z, wdp_ref[...], preferred_element_type=jnp.float32) + bdp_ref[...]
    m = jnp.max(c, axis=1, keepdims=True)
    lse = jnp.log(jnp.sum(jnp.exp(c - m), axis=1, keepdims=True))
    out_ref[...] = c - m - lse


_TC_ROWS = 1000


def kernel(x, edge_index, edge_weight, Wg1, Wg2, Wg3, W1, b1, W3, b3, Wd, bd):
    row = edge_index[0].astype(jnp.int32)
    col = edge_index[1].astype(jnp.int32)
    w = edge_weight.astype(jnp.float32)

    pad = E_PAD - E
    row_p = jnp.concatenate([row, jnp.zeros((pad,), jnp.int32)]).reshape(NW, NCH, K)
    col_p = jnp.concatenate([col, jnp.zeros((pad,), jnp.int32)]).reshape(NW, NCH, K)
    w_p = jnp.concatenate([w, jnp.zeros((pad,), jnp.float32)]).reshape(NW, NCH, K)
    zeros_blk = jnp.zeros((RB, D), jnp.float32)

    ax_p = _sc_agg(x, row_p, col_p, w_p, zeros_blk)

    grid_rows = N // _TC_ROWS
    full = lambda i: (0, 0)
    rows_spec = pl.BlockSpec((_TC_ROWS, D), lambda i: (i, 0))
    part_spec = pl.BlockSpec((NC, _TC_ROWS, D), lambda i: (0, i, 0))
    wspec = pl.BlockSpec((D, D), full)
    bspec = pl.BlockSpec((1, D), full)

    h1, hg2, hl = pl.pallas_call(
        _tc_mid_body,
        grid=(grid_rows,),
        in_specs=[part_spec, rows_spec, wspec, wspec, wspec, bspec],
        out_specs=[rows_spec, rows_spec, rows_spec],
        out_shape=[jax.ShapeDtypeStruct((N, D), jnp.float32)] * 3,
    )(ax_p, x, Wg1, Wg3, W1, b1.reshape(1, D))

    ah_p = _sc_agg(h1, row_p, col_p, w_p, zeros_blk)

    wd_pad = jnp.zeros((D, D), jnp.float32).at[:, :OUT].set(Wd)
    bd_pad = jnp.full((1, D), -1e30, jnp.float32).at[0, :OUT].set(bd)

    out = pl.pallas_call(
        _tc_final_body,
        grid=(grid_rows,),
        in_specs=[part_spec, rows_spec, rows_spec, wspec,
                  pl.BlockSpec((3 * D, D), full), bspec, wspec, bspec],
        out_specs=rows_spec,
        out_shape=jax.ShapeDtypeStruct((N, D), jnp.float32),
    )(ah_p, hg2, hl, Wg2, W3, b3.reshape(1, D), wd_pad, bd_pad)

    return out[:, :OUT]


# speedup vs baseline: 4.5492x; 1.2466x over previous
import functools

import jax
import jax.numpy as jnp
from jax import lax
from jax.experimental import pallas as pl
from jax.experimental.pallas import tpu as pltpu
from jax.experimental.pallas import tpu_sc as plsc

N = 10000
E = 320000
D = 128
OUT = 7

NC = 2
NS = 16
NW = NC * NS
K = 128
NCH = 80
E_PAD = NW * NCH * K
RB = 80
NBLK = N // RB

_MESH = plsc.VectorSubcoreMesh(core_axis_name="c", subcore_axis_name="s")


def _scale_rows(rows_v, w_v):
    def scale(g, carry):
        wv = w_v[pl.ds(g * 16, 16)]
        base = g * 16
        for j in range(16):
            wk = wv[j]
            for d in range(D // 16):
                sl = pl.ds(d * 16, 16)
                rows_v[base + j, sl] = rows_v[base + j, sl] * wk
        return carry
    lax.fori_loop(0, K // 16, scale, 0)


def _sc_agg_body(s_hbm, pack_hbm, w_hbm, zeros_hbm, p_hbm,
                 acc_sh, ip0, ip1, w0, w1, rows0, rows1, si0, si1, sg0, sg1):
    cid = lax.axis_index("c")
    sid = lax.axis_index("s")
    wid = cid * NS + sid

    def _fire_idx(c, ip, wv, si):
        pltpu.async_copy(pack_hbm.at[wid, c], ip, si)
        pltpu.async_copy(w_hbm.at[wid, c], wv, si)

    def _wait_idx(ip, wv, si):
        pltpu.make_async_copy(pack_hbm.at[wid, 0], ip, si).wait()
        pltpu.make_async_copy(w_hbm.at[wid, 0], wv, si).wait()

    for i in range((NBLK + NS - 1) // NS):
        blk = sid + i * NS

        @pl.when(blk < NBLK)
        def _zero():
            pltpu.sync_copy(zeros_hbm, acc_sh.at[pl.ds(blk * RB, RB)])

    _fire_idx(0, ip0, w0, si0)
    _fire_idx(1, ip1, w1, si1)
    plsc.subcore_barrier()
    _wait_idx(ip0, w0, si0)
    pltpu.async_copy(s_hbm.at[ip0.at[1]], rows0, sg0)

    def _slot(c, ipA, wA, rowsA, sgA, siA, ipB, wB, rowsB, sgB, siB):
        @pl.when(c + 1 < NCH)
        def _fire_next_gather():
            _wait_idx(ipB, wB, siB)
            pltpu.async_copy(s_hbm.at[ipB.at[1]], rowsB, sgB)

        pltpu.make_async_copy(s_hbm.at[ipA.at[1]], rowsA, sgA).wait()
        _scale_rows(rowsA, wA)
        pltpu.sync_copy(rowsA, acc_sh.at[ipA.at[0]], add=True)

        @pl.when(c + 2 < NCH)
        def _fire_next_idx():
            _fire_idx(c + 2, ipA, wA, siA)

    def pipe(i, carry):
        c = 2 * i
        _slot(c, ip0, w0, rows0, sg0, si0, ip1, w1, rows1, sg1, si1)
        _slot(c + 1, ip1, w1, rows1, sg1, si1, ip0, w0, rows0, sg0, si0)
        return carry
    lax.fori_loop(0, NCH // 2, pipe, 0)
    plsc.subcore_barrier()

    for i in range((NBLK + NS - 1) // NS):
        blk = sid + i * NS

        @pl.when(blk < NBLK)
        def _writeback():
            r0 = blk * RB
            pltpu.sync_copy(acc_sh.at[pl.ds(r0, RB)], p_hbm.at[cid, pl.ds(r0, RB)])


_sc_agg = functools.partial(
    pl.kernel,
    out_type=jax.ShapeDtypeStruct((NC, N, D), jnp.float32),
    mesh=_MESH,
    scratch_types=[
        pltpu.VMEM_SHARED((N, D), jnp.float32),
        pltpu.VMEM((2, K), jnp.int32),
        pltpu.VMEM((2, K), jnp.int32),
        pltpu.VMEM((K,), jnp.float32),
        pltpu.VMEM((K,), jnp.float32),
        pltpu.VMEM((K, D), jnp.float32),
        pltpu.VMEM((K, D), jnp.float32),
        pltpu.SemaphoreType.DMA,
        pltpu.SemaphoreType.DMA,
        pltpu.SemaphoreType.DMA,
        pltpu.SemaphoreType.DMA,
    ],
)(_sc_agg_body)


def _tc_mid_body(p_ref, x_ref, wg1_ref, wg3_ref, w1_ref, b1_ref,
                 h1_ref, hg2_ref, hl_ref):
    axc = p_ref[0] + p_ref[1]
    h1_ref[...] = jnp.maximum(
        jnp.dot(axc, wg1_ref[...], preferred_element_type=jnp.float32), 0.0)
    hg2_ref[...] = jnp.maximum(
        jnp.dot(axc, wg3_ref[...], preferred_element_type=jnp.float32), 0.0)
    hl_ref[...] = jnp.maximum(
        jnp.dot(x_ref[...], w1_ref[...], preferred_element_type=jnp.float32)
        + b1_ref[...], 0.0)


def _tc_final_body(p_ref, hg2_ref, hl_ref, wg2_ref, w3_ref, b3_ref,
                   wdp_ref, bdp_ref, out_ref):
    ahc = p_ref[0] + p_ref[1]
    hg1 = jnp.maximum(
        jnp.dot(ahc, wg2_ref[...], preferred_element_type=jnp.float32), 0.0)
    z = (jnp.dot(hg1, w3_ref[0:D, :], preferred_element_type=jnp.float32)
         + jnp.dot(hg2_ref[...], w3_ref[D:2 * D, :],
                   preferred_element_type=jnp.float32)
         + jnp.dot(hl_ref[...], w3_ref[2 * D:3 * D, :],
                   preferred_element_type=jnp.float32)
         + b3_ref[...])
    c = jnp.dot(z, wdp_ref[...], preferred_element_type=jnp.float32) + bdp_ref[...]
    m = jnp.max(c, axis=1, keepdims=True)
    lse = jnp.log(jnp.sum(jnp.exp(c - m), axis=1, keepdims=True))
    out_ref[...] = c - m - lse


_TC_ROWS = 1000


def kernel(x, edge_index, edge_weight, Wg1, Wg2, Wg3, W1, b1, W3, b3, Wd, bd):
    row = edge_index[0].astype(jnp.int32)
    col = edge_index[1].astype(jnp.int32)
    w = edge_weight.astype(jnp.float32)

    pad = E_PAD - E
    zpad = jnp.zeros((pad,), jnp.int32)
    row_p = jnp.concatenate([row, zpad]).reshape(NW, NCH, K)
    col_p = jnp.concatenate([col, zpad]).reshape(NW, NCH, K)
    pack = jnp.stack([row_p, col_p], axis=2)
    w_p = jnp.concatenate([w, jnp.zeros((pad,), jnp.float32)]).reshape(NW, NCH, K)
    zeros_blk = jnp.zeros((RB, D), jnp.float32)

    ax_p = _sc_agg(x, pack, w_p, zeros_blk)

    grid_rows = N // _TC_ROWS
    full = lambda i: (0, 0)
    rows_spec = pl.BlockSpec((_TC_ROWS, D), lambda i: (i, 0))
    part_spec = pl.BlockSpec((NC, _TC_ROWS, D), lambda i: (0, i, 0))
    wspec = pl.BlockSpec((D, D), full)
    bspec = pl.BlockSpec((1, D), full)

    h1, hg2, hl = pl.pallas_call(
        _tc_mid_body,
        grid=(grid_rows,),
        in_specs=[part_spec, rows_spec, wspec, wspec, wspec, bspec],
        out_specs=[rows_spec, rows_spec, rows_spec],
        out_shape=[jax.ShapeDtypeStruct((N, D), jnp.float32)] * 3,
    )(ax_p, x, Wg1, Wg3, W1, b1.reshape(1, D))

    ah_p = _sc_agg(h1, pack, w_p, zeros_blk)

    wd_pad = jnp.zeros((D, D), jnp.float32).at[:, :OUT].set(Wd)
    bd_pad = jnp.full((1, D), -1e30, jnp.float32).at[0, :OUT].set(bd)

    out = pl.pallas_call(
        _tc_final_body,
        grid=(grid_rows,),
        in_specs=[part_spec, rows_spec, rows_spec, wspec,
                  pl.BlockSpec((3 * D, D), full), bspec, wspec, bspec],
        out_specs=rows_spec,
        out_shape=jax.ShapeDtypeStruct((N, D), jnp.float32),
    )(ah_p, hg2, hl, Wg2, W3, b3.reshape(1, D), wd_pad, bd_pad)

    return out[:, :OUT]
